# baseline (speedup 1.0000x reference)
import functools

import jax
import jax.numpy as jnp
from jax import lax
from jax.experimental import pallas as pl
from jax.experimental.pallas import tpu as pltpu
from jax.experimental.pallas import tpu_sc as plsc

D = 64
B = 16384
NC = 2
NS = 16
LANES = 16
NW = NC * NS
BPW = B // NW
CHUNK = 128
NCHUNK = BPW // CHUNK
GROUPS = BPW // LANES

_mesh = plsc.VectorSubcoreMesh(
    core_axis_name="c", subcore_axis_name="s", num_cores=NC, num_subcores=NS
)


@functools.partial(
    pl.kernel,
    out_type=jax.ShapeDtypeStruct((B * 2,), jnp.float32),
    mesh=_mesh,
    scratch_types=[
        pltpu.VMEM((NCHUNK, CHUNK), jnp.int32),
        pltpu.VMEM((BPW, D), jnp.float32),
        pltpu.VMEM((BPW * 2,), jnp.float32),
        pltpu.VMEM((2, D, LANES), jnp.float32),
        pltpu.VMEM((2, LANES), jnp.float32),
        pltpu.SemaphoreType.DMA,
    ],
    compiler_params=pltpu.CompilerParams(
        needs_layout_passes=False, use_tc_tiling_on_sc=False
    ),
)
def _router_sc(tok_hbm, table_hbm, w_hbm, b_hbm, out_hbm,
               idx_v, rows_v, out_v, w_v, b_v, sem):
    wid = lax.axis_index("s") * NC + lax.axis_index("c")

    pltpu.sync_copy(tok_hbm.at[wid], idx_v)
    pltpu.sync_copy(w_hbm, w_v)
    pltpu.sync_copy(b_hbm, b_v)

    copies = [
        pltpu.async_copy(
            table_hbm.at[idx_v.at[j]],
            rows_v.at[pl.ds(j * CHUNK, CHUNK)],
            sem,
        )
        for j in range(NCHUNK)
    ]
    for c in copies:
        c.wait()

    iota = lax.iota(jnp.int32, LANES)
    b0 = b_v[0]
    b1 = b_v[1]

    def group(g, carry):
        row_idx = g * LANES + iota

        def dstep(d, accs):
            a0, a1 = accs
            col = plsc.load_gather(rows_v, [row_idx, jnp.full((LANES,), d, jnp.int32)])
            return (a0 + col * w_v[0, d], a1 + col * w_v[1, d])

        a0, a1 = lax.fori_loop(
            0, D, dstep,
            (jnp.zeros((LANES,), jnp.float32), jnp.zeros((LANES,), jnp.float32)),
            unroll=16,
        )
        out_base = row_idx * 2
        plsc.store_scatter(out_v, [out_base], a0 + b0)
        plsc.store_scatter(out_v, [out_base + 1], a1 + b1)
        return carry

    lax.fori_loop(0, GROUPS, group, 0)

    pltpu.sync_copy(out_v, out_hbm.at[pl.ds(wid * BPW * 2, BPW * 2)])


def kernel(ids, token_emb, fc_w, fc_b):
    tok = ids[:, 0].astype(jnp.int32).reshape(NW, NCHUNK, CHUNK)
    w_bcast = jnp.broadcast_to(fc_w[:, :, None], (2, D, LANES))
    b_bcast = jnp.broadcast_to(fc_b[:, None], (2, LANES))
    return _router_sc(tok, token_emb, w_bcast, b_bcast).reshape(B, 2)
